# baseline (speedup 1.0000x reference)
import functools

import jax
import jax.numpy as jnp
import numpy as np
from jax import lax
from jax.experimental import pallas as pl
from jax.experimental.pallas import tpu as pltpu
from jax.experimental.pallas import tpu_sc as plsc

_SWAP_PROB = 0.1

_NC = 2
_NS = 16
_NW = _NC * _NS
_LANES = 16

_CHUNK_ROWS = 8
_NBUF = 8


def _ceil16(n):
    return (int(n) + 15) // 16 * 16


_ROTS = ((13, 15, 26, 6), (17, 29, 16, 24))


def _rotl(x, d):
    return (x << np.uint32(d)) | (x >> np.uint32(32 - d))


def _threefry2x32(k0, k1, x0, x1):
    k0, k1 = np.uint32(k0), np.uint32(k1)
    k2 = k0 ^ k1 ^ np.uint32(0x1BD11BDA)
    ks = (k0, k1, k2)
    x0 = (x0 + k0).astype(np.uint32)
    x1 = (x1 + k1).astype(np.uint32)
    for i in range(5):
        for r in _ROTS[i % 2]:
            x0 = (x0 + x1).astype(np.uint32)
            x1 = _rotl(x1, r) ^ x0
        x0 = (x0 + ks[(i + 1) % 3]).astype(np.uint32)
        x1 = (x1 + ks[(i + 2) % 3] + np.uint32(i + 1)).astype(np.uint32)
    return x0, x1


def _np_uniform(k0, k1, shape):
    n = int(np.prod(shape))
    idx = np.arange(n, dtype=np.uint64)
    c1 = (idx >> np.uint64(32)).astype(np.uint32)
    c2 = (idx & np.uint64(0xFFFFFFFF)).astype(np.uint32)
    b1, b2 = _threefry2x32(k0, k1, c1, c2)
    bits = b1 ^ b2
    fb = (bits >> np.uint32(9)) | np.uint32(0x3F800000)
    u = fb.view(np.float32) - np.float32(1.0)
    return np.maximum(np.float32(0.0), u).reshape(shape)


def _np_mask_perm(B, F):
    idx = np.arange(2, dtype=np.uint64)
    b1, b2 = _threefry2x32(
        np.uint32(0), np.uint32(42),
        (idx >> np.uint64(32)).astype(np.uint32),
        (idx & np.uint64(0xFFFFFFFF)).astype(np.uint32))
    mask = _np_uniform(b1[0], b2[0], (B, F)) < np.float32(_SWAP_PROB)
    perm = np.argsort(_np_uniform(b1[1], b2[1], (B, F)), axis=1, kind="stable")
    return mask, perm


@functools.lru_cache(maxsize=2)
def _bake(B, F):
    mask_np, perm_np = _np_mask_perm(B, F)
    perm_np = perm_np.astype(np.int64)

    CH = _CHUNK_ROWS
    n_chunks_total = B // CH

    r_l = np.arange(CH)[:, None]
    c_l = np.arange(F)[None, :]
    P = (r_l * F + c_l).astype(np.int64)

    i_idx, j_idx = np.nonzero(mask_np)
    src_col = perm_np[i_idx, j_idx]
    conf = mask_np[i_idx, src_col]
    chunk = i_idx // CH
    dst_p = P[i_idx % CH, j_idx]
    src_p = P[i_idx % CH, src_col]

    mask_phys = np.zeros((n_chunks_total, CH * F), dtype=bool)
    mask_phys[chunk, dst_p] = True
    pp = np.argmin(mask_phys, axis=1)

    span = np.int64(2 * CH * F)
    ukeys = np.unique(chunk[conf] * span + src_p[conf])
    uchunk = ukeys // span
    usrc = ukeys % span
    spill_counts = np.bincount(uchunk, minlength=n_chunks_total)
    s_pad = _ceil16(spill_counts.max() + 1)
    spill_start = np.concatenate([[0], np.cumsum(spill_counts)])
    uslot = np.arange(len(ukeys)) - spill_start[uchunk] + 1

    s1 = np.repeat(pp[:, None], s_pad, axis=1)
    s1[:, 0] = pp
    s1[uchunk, uslot] = usrc

    pos = np.searchsorted(ukeys, chunk * span + src_p)
    slot_of_pair = np.where(conf, np.take(uslot, np.minimum(pos, len(ukeys) - 1),
                                          mode='clip'), 0)
    p1_counts = np.bincount(chunk[~conf], minlength=n_chunks_total)
    p2_counts = np.bincount(chunk[conf], minlength=n_chunks_total)
    p1_pad = _ceil16(p1_counts.max())
    p2_pad = _ceil16(p2_counts.max())
    p1_pad += (-(s_pad + p1_pad + p2_pad)) % 128

    pad1 = (pp | (pp << 16)).astype(np.int64)
    s2 = np.repeat(pad1[:, None], p1_pad, axis=1)
    c1 = chunk[~conf]
    st1 = np.concatenate([[0], np.cumsum(p1_counts)])
    ppos1 = np.arange(len(c1)) - st1[c1]
    s2[c1, ppos1] = dst_p[~conf] | (src_p[~conf] << 16)

    pad2 = pp.astype(np.int64)
    s3 = np.repeat(pad2[:, None], p2_pad, axis=1)
    c2 = chunk[conf]
    st2 = np.concatenate([[0], np.cumsum(p2_counts)])
    ppos2 = np.arange(len(c2)) - st2[c2]
    s3[c2, ppos2] = dst_p[conf] | (slot_of_pair[conf] << 16)

    pv = np.concatenate([s1, s2, s3], axis=1).reshape(-1).astype(np.int32)
    return jnp.asarray(pv), int(s_pad), int(p1_pad), int(p2_pad)


def _make_sc_swap(B, F, s_pad, p1_pad, p2_pad):
    CH = _CHUNK_ROWS
    rows_per_w = B // _NW
    n_chunks = rows_per_w // CH
    blk = s_pad + p1_pad + p2_pad
    spill_len = -(-s_pad // 128) * 128
    assert n_chunks % _NBUF == 0 and n_chunks >= 2 * _NBUF
    assert F == 1024

    mesh = plsc.VectorSubcoreMesh(core_axis_name="c", subcore_axis_name="s")

    @functools.partial(
        pl.kernel,
        out_type=jax.ShapeDtypeStruct((B, F), jnp.float32),
        mesh=mesh,
        compiler_params=pltpu.CompilerParams(
            needs_layout_passes=False, use_tc_tiling_on_sc=True),
        scratch_types=(
            [pltpu.VMEM((CH, F), jnp.float32)] * _NBUF
            + [pltpu.VMEM((blk,), jnp.int32)] * _NBUF
            + [pltpu.VMEM((spill_len,), jnp.float32),
               pltpu.SemaphoreType.DMA((_NBUF,)),
               pltpu.SemaphoreType.DMA((_NBUF,))]
        ),
    )
    def sc_swap(x_hbm, pv_hbm, out_hbm, *scratch):
        ovs = scratch[:_NBUF]
        pvs = scratch[_NBUF:2 * _NBUF]
        sp = scratch[2 * _NBUF]
        in_sem, out_sem = scratch[2 * _NBUF + 1], scratch[2 * _NBUF + 2]
        wid = lax.axis_index("s") * _NC + lax.axis_index("c")
        rbase = wid * rows_per_w
        pbase = wid * n_chunks * blk

        def start_in(c, b):
            pltpu.async_copy(
                x_hbm.at[pl.ds(rbase + c * CH, CH)],
                ovs[b].at[pl.ds(0, CH)], in_sem.at[b])
            pltpu.async_copy(
                pv_hbm.at[pl.ds(pbase + c * blk, blk)],
                pvs[b], in_sem.at[b])

        def wait_in(b):
            pltpu.make_async_copy(
                x_hbm.at[pl.ds(0, CH)],
                ovs[b].at[pl.ds(0, CH)], in_sem.at[b]).wait()
            pltpu.make_async_copy(
                pv_hbm.at[pl.ds(0, blk)], pvs[b], in_sem.at[b]).wait()

        def start_out(c, b):
            pltpu.async_copy(
                ovs[b].at[pl.ds(0, CH)],
                out_hbm.at[pl.ds(rbase + c * CH, CH)], out_sem.at[b])

        def wait_out(b):
            pltpu.make_async_copy(
                ovs[b].at[pl.ds(0, CH)],
                out_hbm.at[pl.ds(0, CH)], out_sem.at[b]).wait()

        def process(c, b):
            wait_in(b)
            ob, pb = ovs[b], pvs[b]

            @plsc.parallel_loop(0, s_pad // _LANES, unroll=2)
            def stage1(k):
                w = pb[pl.ds(k * _LANES, _LANES)]
                vals = plsc.load_gather(ob, [w >> 10, w & 0x3FF])
                sp[pl.ds(k * _LANES, _LANES)] = vals

            @plsc.parallel_loop(0, p1_pad // _LANES, unroll=4)
            def stage2(k):
                w = pb[pl.ds(s_pad + k * _LANES, _LANES)]
                s = w >> 16
                d = w & 0xFFFF
                vals = plsc.load_gather(ob, [s >> 10, s & 0x3FF])
                plsc.store_scatter(ob, [d >> 10, d & 0x3FF], vals)

            @plsc.parallel_loop(0, p2_pad // _LANES, unroll=2)
            def stage3(k):
                w = pb[pl.ds(s_pad + p1_pad + k * _LANES, _LANES)]
                d = w & 0xFFFF
                vals = plsc.load_gather(sp, [w >> 16])
                plsc.store_scatter(ob, [d >> 10, d & 0x3FF], vals)

            start_out(c, b)

        for b in range(_NBUF):
            start_in(b, b)
        process(0, 0)

        def outer(o, _):
            for b in range(_NBUF):
                c = o * _NBUF + b + 1
                wait_out(b)
                start_in(c + _NBUF - 1, b)
                process(c, (b + 1) % _NBUF)
            return 0

        lax.fori_loop(0, (n_chunks - _NBUF) // _NBUF, outer, 0)
        for c in range(n_chunks - _NBUF + 1, n_chunks):
            process(c, c % _NBUF)
        for b in range(_NBUF):
            wait_out(b)

    return sc_swap


def kernel(x):
    B, F = x.shape
    pv, s_pad, p1_pad, p2_pad = _bake(B, F)
    return _make_sc_swap(B, F, s_pad, p1_pad, p2_pad)(x, pv)


# speedup vs baseline: 1.0387x; 1.0387x over previous
import functools

import jax
import jax.numpy as jnp
import numpy as np
from jax import lax
from jax.experimental import pallas as pl
from jax.experimental.pallas import tpu as pltpu
from jax.experimental.pallas import tpu_sc as plsc

_SWAP_PROB = 0.1

_NC = 2
_NS = 16
_NW = _NC * _NS
_LANES = 16

_CHUNK_ROWS = 16
_NBUF = 4


def _ceil16(n):
    return (int(n) + 15) // 16 * 16


_ROTS = ((13, 15, 26, 6), (17, 29, 16, 24))


def _rotl(x, d):
    return (x << np.uint32(d)) | (x >> np.uint32(32 - d))


def _threefry2x32(k0, k1, x0, x1):
    k0, k1 = np.uint32(k0), np.uint32(k1)
    k2 = k0 ^ k1 ^ np.uint32(0x1BD11BDA)
    ks = (k0, k1, k2)
    x0 = (x0 + k0).astype(np.uint32)
    x1 = (x1 + k1).astype(np.uint32)
    for i in range(5):
        for r in _ROTS[i % 2]:
            x0 = (x0 + x1).astype(np.uint32)
            x1 = _rotl(x1, r) ^ x0
        x0 = (x0 + ks[(i + 1) % 3]).astype(np.uint32)
        x1 = (x1 + ks[(i + 2) % 3] + np.uint32(i + 1)).astype(np.uint32)
    return x0, x1


def _np_uniform(k0, k1, shape):
    n = int(np.prod(shape))
    idx = np.arange(n, dtype=np.uint64)
    c1 = (idx >> np.uint64(32)).astype(np.uint32)
    c2 = (idx & np.uint64(0xFFFFFFFF)).astype(np.uint32)
    b1, b2 = _threefry2x32(k0, k1, c1, c2)
    bits = b1 ^ b2
    fb = (bits >> np.uint32(9)) | np.uint32(0x3F800000)
    u = fb.view(np.float32) - np.float32(1.0)
    return np.maximum(np.float32(0.0), u).reshape(shape)


def _np_mask_perm(B, F):
    idx = np.arange(2, dtype=np.uint64)
    b1, b2 = _threefry2x32(
        np.uint32(0), np.uint32(42),
        (idx >> np.uint64(32)).astype(np.uint32),
        (idx & np.uint64(0xFFFFFFFF)).astype(np.uint32))
    mask = _np_uniform(b1[0], b2[0], (B, F)) < np.float32(_SWAP_PROB)
    perm = np.argsort(_np_uniform(b1[1], b2[1], (B, F)), axis=1, kind="stable")
    return mask, perm


@functools.lru_cache(maxsize=2)
def _bake(B, F):
    mask_np, perm_np = _np_mask_perm(B, F)
    perm_np = perm_np.astype(np.int64)

    CH = _CHUNK_ROWS
    n_chunks_total = B // CH

    r_l = np.arange(CH)[:, None]
    c_l = np.arange(F)[None, :]
    P = (r_l * F + c_l).astype(np.int64)

    i_idx, j_idx = np.nonzero(mask_np)
    src_col = perm_np[i_idx, j_idx]
    conf = mask_np[i_idx, src_col]
    chunk = i_idx // CH
    dst_p = P[i_idx % CH, j_idx]
    src_p = P[i_idx % CH, src_col]

    mask_phys = np.zeros((n_chunks_total, CH * F), dtype=bool)
    mask_phys[chunk, dst_p] = True
    pp = np.argmin(mask_phys, axis=1)

    span = np.int64(2 * CH * F)
    ukeys = np.unique(chunk[conf] * span + src_p[conf])
    uchunk = ukeys // span
    usrc = ukeys % span
    spill_counts = np.bincount(uchunk, minlength=n_chunks_total)
    s_pad = _ceil16(spill_counts.max() + 1)
    spill_start = np.concatenate([[0], np.cumsum(spill_counts)])
    uslot = np.arange(len(ukeys)) - spill_start[uchunk] + 1

    s1 = np.repeat(pp[:, None], s_pad, axis=1)
    s1[:, 0] = pp
    s1[uchunk, uslot] = usrc

    pos = np.searchsorted(ukeys, chunk * span + src_p)
    slot_of_pair = np.where(conf, np.take(uslot, np.minimum(pos, len(ukeys) - 1),
                                          mode='clip'), 0)
    p1_counts = np.bincount(chunk[~conf], minlength=n_chunks_total)
    p2_counts = np.bincount(chunk[conf], minlength=n_chunks_total)
    p1_pad = _ceil16(p1_counts.max())
    p2_pad = _ceil16(p2_counts.max())
    p1_pad += (-(s_pad + p1_pad + p2_pad)) % 128

    pad1 = (pp | (pp << 16)).astype(np.int64)
    s2 = np.repeat(pad1[:, None], p1_pad, axis=1)
    c1 = chunk[~conf]
    st1 = np.concatenate([[0], np.cumsum(p1_counts)])
    ppos1 = np.arange(len(c1)) - st1[c1]
    s2[c1, ppos1] = dst_p[~conf] | (src_p[~conf] << 16)

    pad2 = pp.astype(np.int64)
    s3 = np.repeat(pad2[:, None], p2_pad, axis=1)
    c2 = chunk[conf]
    st2 = np.concatenate([[0], np.cumsum(p2_counts)])
    ppos2 = np.arange(len(c2)) - st2[c2]
    s3[c2, ppos2] = dst_p[conf] | (slot_of_pair[conf] << 16)

    pv = np.concatenate([s1, s2, s3], axis=1).reshape(-1).astype(np.int32)
    return jnp.asarray(pv), int(s_pad), int(p1_pad), int(p2_pad)


def _make_sc_swap(B, F, s_pad, p1_pad, p2_pad):
    CH = _CHUNK_ROWS
    rows_per_w = B // _NW
    n_chunks = rows_per_w // CH
    blk = s_pad + p1_pad + p2_pad
    spill_len = -(-s_pad // 128) * 128
    assert n_chunks % _NBUF == 0 and n_chunks >= 2 * _NBUF
    assert F == 1024

    mesh = plsc.VectorSubcoreMesh(core_axis_name="c", subcore_axis_name="s")

    @functools.partial(
        pl.kernel,
        out_type=jax.ShapeDtypeStruct((B, F), jnp.float32),
        mesh=mesh,
        compiler_params=pltpu.CompilerParams(
            needs_layout_passes=False, use_tc_tiling_on_sc=True),
        scratch_types=(
            [pltpu.VMEM((CH, F), jnp.float32)] * _NBUF
            + [pltpu.VMEM((blk,), jnp.int32)] * _NBUF
            + [pltpu.VMEM((spill_len,), jnp.float32),
               pltpu.SemaphoreType.DMA((_NBUF,)),
               pltpu.SemaphoreType.DMA((_NBUF,))]
        ),
    )
    def sc_swap(x_hbm, pv_hbm, out_hbm, *scratch):
        ovs = scratch[:_NBUF]
        pvs = scratch[_NBUF:2 * _NBUF]
        sp = scratch[2 * _NBUF]
        in_sem, out_sem = scratch[2 * _NBUF + 1], scratch[2 * _NBUF + 2]
        wid = lax.axis_index("s") * _NC + lax.axis_index("c")
        rbase = wid * rows_per_w
        pbase = wid * n_chunks * blk

        def start_in(c, b):
            pltpu.async_copy(
                x_hbm.at[pl.ds(rbase + c * CH, CH)],
                ovs[b].at[pl.ds(0, CH)], in_sem.at[b])
            pltpu.async_copy(
                pv_hbm.at[pl.ds(pbase + c * blk, blk)],
                pvs[b], in_sem.at[b])

        def wait_in(b):
            pltpu.make_async_copy(
                x_hbm.at[pl.ds(0, CH)],
                ovs[b].at[pl.ds(0, CH)], in_sem.at[b]).wait()
            pltpu.make_async_copy(
                pv_hbm.at[pl.ds(0, blk)], pvs[b], in_sem.at[b]).wait()

        def start_out(c, b):
            pltpu.async_copy(
                ovs[b].at[pl.ds(0, CH)],
                out_hbm.at[pl.ds(rbase + c * CH, CH)], out_sem.at[b])

        def wait_out(b):
            pltpu.make_async_copy(
                ovs[b].at[pl.ds(0, CH)],
                out_hbm.at[pl.ds(0, CH)], out_sem.at[b]).wait()

        def process(c, b):
            wait_in(b)
            ob, pb = ovs[b], pvs[b]

            @plsc.parallel_loop(0, s_pad // _LANES, unroll=2)
            def stage1(k):
                w = pb[pl.ds(k * _LANES, _LANES)]
                vals = plsc.load_gather(ob, [w >> 10, w & 0x3FF])
                sp[pl.ds(k * _LANES, _LANES)] = vals

            @plsc.parallel_loop(0, p1_pad // _LANES, unroll=4)
            def stage2(k):
                w = pb[pl.ds(s_pad + k * _LANES, _LANES)]
                s = w >> 16
                d = w & 0xFFFF
                vals = plsc.load_gather(ob, [s >> 10, s & 0x3FF])
                plsc.store_scatter(ob, [d >> 10, d & 0x3FF], vals)

            @plsc.parallel_loop(0, p2_pad // _LANES, unroll=2)
            def stage3(k):
                w = pb[pl.ds(s_pad + p1_pad + k * _LANES, _LANES)]
                d = w & 0xFFFF
                vals = plsc.load_gather(sp, [w >> 16])
                plsc.store_scatter(ob, [d >> 10, d & 0x3FF], vals)

            start_out(c, b)

        for b in range(_NBUF):
            start_in(b, b)
        process(0, 0)

        def outer(o, _):
            for b in range(_NBUF):
                c = o * _NBUF + b + 1
                wait_out(b)
                start_in(c + _NBUF - 1, b)
                process(c, (b + 1) % _NBUF)
            return 0

        lax.fori_loop(0, (n_chunks - _NBUF) // _NBUF, outer, 0)
        for c in range(n_chunks - _NBUF + 1, n_chunks):
            process(c, c % _NBUF)
        for b in range(_NBUF):
            wait_out(b)

    return sc_swap


def kernel(x):
    B, F = x.shape
    pv, s_pad, p1_pad, p2_pad = _bake(B, F)
    return _make_sc_swap(B, F, s_pad, p1_pad, p2_pad)(x, pv)
